# baseline (speedup 1.0000x reference)
import functools

import jax
import jax.numpy as jnp
from jax import lax
from jax.experimental import pallas as pl
from jax.experimental.pallas import tpu as pltpu
from jax.experimental.pallas import tpu_sc as plsc

B = 16384
F = 26
V = 1000000
D = 16
H1 = 400
H2 = 400

N = B * F
NC = 2
NS = 16
NW = NC * NS
N_PER_W = N // NW
CHUNK = 1664
N_CHUNKS = N_PER_W // CHUNK


TBLK = 2048
T_GRID = -(-V // TBLK)


def _tc_transpose_body(xt_ref, out_ref):
    x = xt_ref[...]
    y = jnp.transpose(x)
    y3 = y.reshape(TBLK // 8, 8, D)
    out_ref[...] = jnp.concatenate([y3[:, k, :] for k in range(8)], axis=-1)


def _tc_transpose(emb_t):
    return pl.pallas_call(
        _tc_transpose_body,
        grid=(T_GRID,),
        in_specs=[pl.BlockSpec((D, TBLK), lambda i: (0, i))],
        out_specs=pl.BlockSpec((TBLK // 8, 128), lambda i: (i, 0)),
        out_shape=jax.ShapeDtypeStruct((V // 8, 128), jnp.float32),
    )(emb_t)


def _sc_gather_body(idx_hbm, emb_hbm, lin_hbm, emb_out, lin_out,
                    idx_v, emb_v, lin_v, sem_e, sem_l):
    wid = lax.axis_index("s") * NC + lax.axis_index("c")
    base = wid * N_PER_W
    for c in range(N_CHUNKS):
        off = base + c * CHUNK
        pltpu.sync_copy(idx_hbm.at[pl.ds(off, CHUNK)], idx_v)
        cp_e = pltpu.async_copy(emb_hbm.at[idx_v], emb_v, sem_e)
        cp_l = pltpu.async_copy(lin_hbm.at[idx_v], lin_v, sem_l)
        cp_e.wait()
        cp_l.wait()
        pltpu.sync_copy(emb_v, emb_out.at[pl.ds(off, CHUNK)])
        pltpu.sync_copy(lin_v, lin_out.at[pl.ds(off, CHUNK)])


@functools.partial(jax.jit, donate_argnums=())
def _sc_gather(idx_flat, emb_table, lin_flat):
    mesh = plsc.VectorSubcoreMesh(core_axis_name="c", subcore_axis_name="s")
    return pl.kernel(
        _sc_gather_body,
        out_type=[
            jax.ShapeDtypeStruct((N, D), jnp.float32),
            jax.ShapeDtypeStruct((N,), jnp.float32),
        ],
        name="deepfm_sc_gather",
        mesh=mesh,
        compiler_params=pltpu.CompilerParams(use_tc_tiling_on_sc=False),
        scratch_types=[
            pltpu.VMEM((CHUNK,), jnp.int32),
            pltpu.VMEM((CHUNK, D), jnp.float32),
            pltpu.VMEM((CHUNK,), jnp.float32),
            pltpu.SemaphoreType.DMA,
            pltpu.SemaphoreType.DMA,
        ],
    )(idx_flat, emb_table, lin_flat)


BLK = 1024


def _tc_body(emb_ref, lin_ref, w1_ref, b1_ref, w2_ref, b2_ref, w3_ref,
             b3_ref, s_ref, out_ref):
    emb = emb_ref[...]
    s = s_ref[...]
    sum_emb = jnp.dot(emb, s, preferred_element_type=jnp.float32)
    sum_sq = jnp.dot(emb * emb, s, preferred_element_type=jnp.float32)
    fm = 0.5 * jnp.sum(sum_emb * sum_emb - sum_sq, axis=-1, keepdims=True)
    first = jnp.sum(lin_ref[...], axis=-1, keepdims=True)
    h = jnp.dot(emb, w1_ref[...], preferred_element_type=jnp.float32)
    h = jnp.maximum(h + b1_ref[...], 0.0)
    h = jnp.dot(h, w2_ref[...], preferred_element_type=jnp.float32)
    h = jnp.maximum(h + b2_ref[...], 0.0)
    dnn = jnp.sum(h * w3_ref[...], axis=-1, keepdims=True) + b3_ref[...]
    out_ref[...] = jax.nn.sigmoid(first + fm + dnn)


def _tc_head(emb_flat, lin_vals, W1, b1, W2, b2, W3, b3, s_mat):
    grid = (B // BLK,)
    return pl.pallas_call(
        _tc_body,
        grid=grid,
        in_specs=[
            pl.BlockSpec((BLK, F * D), lambda i: (i, 0)),
            pl.BlockSpec((BLK, F), lambda i: (i, 0)),
            pl.BlockSpec((F * D, H1), lambda i: (0, 0)),
            pl.BlockSpec((1, H1), lambda i: (0, 0)),
            pl.BlockSpec((H1, H2), lambda i: (0, 0)),
            pl.BlockSpec((1, H2), lambda i: (0, 0)),
            pl.BlockSpec((1, H2), lambda i: (0, 0)),
            pl.BlockSpec((1, 1), lambda i: (0, 0)),
            pl.BlockSpec((F * D, D), lambda i: (0, 0)),
        ],
        out_specs=pl.BlockSpec((BLK, 1), lambda i: (i, 0)),
        out_shape=jax.ShapeDtypeStruct((B, 1), jnp.float32),
    )(emb_flat, lin_vals, W1, b1, W2, b2, W3, b3, s_mat)


def kernel(indices, emb_table, lin_table, W1, b1, W2, b2, W3, b3):
    idx_flat = indices.reshape(-1).astype(jnp.int32)
    lin_flat = lin_table.reshape(-1)
    emb_r8 = _tc_transpose(emb_table.T)
    emb_rows, lin_rows = _sc_gather(idx_flat, emb_r8.reshape(V, D), lin_flat)
    emb_flat = emb_rows.reshape(B, F * D)
    lin_vals = lin_rows.reshape(B, F)
    s_mat = jnp.tile(jnp.eye(D, dtype=jnp.float32), (F, 1))
    return _tc_head(emb_flat, lin_vals, W1, b1.reshape(1, H1), W2,
                    b2.reshape(1, H2), W3.reshape(1, H2), b3.reshape(1, 1),
                    s_mat)


# speedup vs baseline: 1.0509x; 1.0509x over previous
import functools

import jax
import jax.numpy as jnp
from jax import lax
from jax.experimental import pallas as pl
from jax.experimental.pallas import tpu as pltpu
from jax.experimental.pallas import tpu_sc as plsc

B = 16384
F = 26
V = 1000000
D = 16
H1 = 400
H2 = 400

NSLICE = 2
BS = B // NSLICE
NS_ROWS = BS * F
NC = 2
NSUB = 16
NW = NC * NSUB
N_PER_W = NS_ROWS // NW
CHUNK = 1664
N_CHUNKS = N_PER_W // CHUNK


def _sc_gather_body(idx_hbm, emb_hbm, lin_hbm, emb_out, lin_out,
                    idx_v, emb_v, lin_v, sem_e, sem_l):
    wid = lax.axis_index("s") * NC + lax.axis_index("c")
    base = wid * N_PER_W
    for c in range(N_CHUNKS):
        off = base + c * CHUNK
        pltpu.sync_copy(idx_hbm.at[pl.ds(off, CHUNK)], idx_v)
        cp_e = pltpu.async_copy(emb_hbm.at[idx_v], emb_v, sem_e)
        cp_l = pltpu.async_copy(lin_hbm.at[idx_v], lin_v, sem_l)
        cp_e.wait()
        cp_l.wait()
        pltpu.sync_copy(emb_v, emb_out.at[pl.ds(off, CHUNK)])
        pltpu.sync_copy(lin_v, lin_out.at[pl.ds(off, CHUNK)])


@functools.partial(jax.jit, donate_argnums=())
def _sc_gather(idx_flat, emb_table, lin_flat):
    mesh = plsc.VectorSubcoreMesh(core_axis_name="c", subcore_axis_name="s")
    return pl.kernel(
        _sc_gather_body,
        out_type=[
            jax.ShapeDtypeStruct((NS_ROWS, D), jnp.float32),
            jax.ShapeDtypeStruct((NS_ROWS,), jnp.float32),
        ],
        name="deepfm_sc_gather",
        mesh=mesh,
        compiler_params=pltpu.CompilerParams(use_tc_tiling_on_sc=False),
        scratch_types=[
            pltpu.VMEM((CHUNK,), jnp.int32),
            pltpu.VMEM((CHUNK, D), jnp.float32),
            pltpu.VMEM((CHUNK,), jnp.float32),
            pltpu.SemaphoreType.DMA,
            pltpu.SemaphoreType.DMA,
        ],
    )(idx_flat, emb_table, lin_flat)


BLK = 1024


def _tc_body(emb_ref, lin_ref, w1_ref, b1_ref, w2_ref, b2_ref, w3_ref,
             b3_ref, s_ref, out_ref):
    emb = emb_ref[...]
    s = s_ref[...]
    sum_emb = jnp.dot(emb, s, preferred_element_type=jnp.float32)
    sum_sq = jnp.dot(emb * emb, s, preferred_element_type=jnp.float32)
    fm = 0.5 * jnp.sum(sum_emb * sum_emb - sum_sq, axis=-1, keepdims=True)
    first = jnp.sum(lin_ref[...], axis=-1, keepdims=True)
    h = jnp.dot(emb, w1_ref[...], preferred_element_type=jnp.float32)
    h = jnp.maximum(h + b1_ref[...], 0.0)
    h = jnp.dot(h, w2_ref[...], preferred_element_type=jnp.float32)
    h = jnp.maximum(h + b2_ref[...], 0.0)
    dnn = jnp.sum(h * w3_ref[...], axis=-1, keepdims=True) + b3_ref[...]
    out_ref[...] = jax.nn.sigmoid(first + fm + dnn)


def _tc_head(emb_flat, lin_vals, W1, b1, W2, b2, W3, b3, s_mat):
    grid = (BS // BLK,)
    return pl.pallas_call(
        _tc_body,
        grid=grid,
        in_specs=[
            pl.BlockSpec((BLK, F * D), lambda i: (i, 0)),
            pl.BlockSpec((BLK, F), lambda i: (i, 0)),
            pl.BlockSpec((F * D, H1), lambda i: (0, 0)),
            pl.BlockSpec((1, H1), lambda i: (0, 0)),
            pl.BlockSpec((H1, H2), lambda i: (0, 0)),
            pl.BlockSpec((1, H2), lambda i: (0, 0)),
            pl.BlockSpec((1, H2), lambda i: (0, 0)),
            pl.BlockSpec((1, 1), lambda i: (0, 0)),
            pl.BlockSpec((F * D, D), lambda i: (0, 0)),
        ],
        out_specs=pl.BlockSpec((BLK, 1), lambda i: (i, 0)),
        out_shape=jax.ShapeDtypeStruct((BS, 1), jnp.float32),
    )(emb_flat, lin_vals, W1, b1, W2, b2, W3, b3, s_mat)


def kernel(indices, emb_table, lin_table, W1, b1, W2, b2, W3, b3):
    idx_flat = indices.reshape(-1).astype(jnp.int32)
    lin_flat = lin_table.reshape(-1)
    emb_r8 = lax.optimization_barrier(emb_table.reshape(V // 8, D * 8))
    emb_lin = emb_r8.reshape(V, D)
    s_mat = jnp.tile(jnp.eye(D, dtype=jnp.float32), (F, 1))
    b1r = b1.reshape(1, H1)
    b2r = b2.reshape(1, H2)
    w3r = W3.reshape(1, H2)
    b3r = b3.reshape(1, 1)
    outs = []
    for s in range(NSLICE):
        idx_s = lax.dynamic_slice(idx_flat, (s * NS_ROWS,), (NS_ROWS,))
        emb_rows, lin_rows = _sc_gather(idx_s, emb_lin, lin_flat)
        emb_flat = emb_rows.reshape(BS, F * D)
        lin_vals = lin_rows.reshape(BS, F)
        outs.append(_tc_head(emb_flat, lin_vals, W1, b1r, W2, b2r, w3r, b3r,
                             s_mat))
    return jnp.concatenate(outs, axis=0)
